# baseline (speedup 1.0000x reference)
import functools

import jax
import jax.numpy as jnp
from jax.experimental import pallas as pl
from jax.experimental.pallas import tpu as pltpu


def _moe_body(x_ref, wg_ref, up_hbm, gate_hbm, down_hbm, out_ref,
              ubuf, gbuf, dbuf, sems):
    E = wg_ref.shape[1]

    xf = x_ref[...].astype(jnp.float32)
    logits = jnp.dot(xf, wg_ref[...], preferred_element_type=jnp.float32)
    m = jnp.max(logits, axis=-1, keepdims=True)
    p = jnp.exp(logits - m)
    g = p / jnp.sum(p, axis=-1, keepdims=True)
    col = jax.lax.broadcasted_iota(jnp.int32, g.shape, 1)
    i1 = jnp.argmax(g, axis=-1)[:, None]
    oh1 = col == i1
    i2 = jnp.argmax(jnp.where(oh1, -1.0, g), axis=-1)[:, None]
    oh2 = col == i2
    w = jnp.where(oh1 | oh2, g, 0.0)

    def copies(e):
        slot = e % 3
        return (
            pltpu.make_async_copy(up_hbm.at[e], ubuf.at[slot], sems.at[0, slot]),
            pltpu.make_async_copy(gate_hbm.at[e], gbuf.at[slot], sems.at[1, slot]),
            pltpu.make_async_copy(down_hbm.at[e], dbuf.at[slot], sems.at[2, slot]),
        )

    for c in copies(0):
        c.start()
    for c in copies(1):
        c.start()
    for c in copies(2):
        c.start()

    xb = x_ref[...]
    dn = (((1,), (1,)), ((), ()))
    H = ubuf.shape[1]
    HC = 512
    NC = H // HC
    wes = [jnp.sum(jnp.where(col == e, w, 0.0), axis=1, keepdims=True)
           for e in range(E)]
    acc = None
    DEPTH = 2
    pend = []
    for k in range(E * NC):
        e, c = divmod(k, NC)
        slot = e % 3
        cs = c * HC
        if c == 0:
            cu, cg, cd = copies(e)
            cu.wait()
            cg.wait()
            cd.wait()
        gate_c = jax.lax.dot_general(xb, gbuf[slot, cs:cs + HC, :], dn,
                                     preferred_element_type=jnp.float32)
        up_c = jax.lax.dot_general(xb, ubuf[slot, cs:cs + HC, :], dn,
                                   preferred_element_type=jnp.float32)
        silu_c = gate_c * jax.nn.sigmoid(gate_c)
        hidden_c = (silu_c * up_c).astype(jnp.bfloat16)
        pend.append((hidden_c, slot, cs, e))
        if len(pend) > DEPTH:
            ph, pslot, pcs, pe = pend.pop(0)
            y_c = jax.lax.dot_general(ph, dbuf[pslot, :, pcs:pcs + HC], dn,
                                      preferred_element_type=jnp.float32)
            contrib = wes[pe] * y_c
            acc = contrib if acc is None else acc + contrib
        if c == DEPTH - 1 and e >= 1 and e + 2 < E:
            for cp in copies(e + 2):
                cp.start()
    for ph, pslot, pcs, pe in pend:
        y_c = jax.lax.dot_general(ph, dbuf[pslot, :, pcs:pcs + HC], dn,
                                  preferred_element_type=jnp.float32)
        acc = acc + wes[pe] * y_c

    out_ref[...] = acc


@jax.jit
def kernel(x, W_gate, up_proj, gate_proj, down_proj):
    S, D = x.shape
    E, H, _ = up_proj.shape
    return pl.pallas_call(
        _moe_body,
        in_specs=[
            pl.BlockSpec(memory_space=pltpu.VMEM),
            pl.BlockSpec(memory_space=pltpu.VMEM),
            pl.BlockSpec(memory_space=pl.ANY),
            pl.BlockSpec(memory_space=pl.ANY),
            pl.BlockSpec(memory_space=pl.ANY),
        ],
        out_specs=pl.BlockSpec(memory_space=pltpu.VMEM),
        out_shape=jax.ShapeDtypeStruct((S, D), jnp.float32),
        scratch_shapes=[
            pltpu.VMEM((3, H, D), jnp.bfloat16),
            pltpu.VMEM((3, H, D), jnp.bfloat16),
            pltpu.VMEM((3, D, H), jnp.bfloat16),
            pltpu.SemaphoreType.DMA((3, 3)),
        ],
    )(x, W_gate, up_proj, gate_proj, down_proj)


# speedup vs baseline: 1.0037x; 1.0037x over previous
import functools

import jax
import jax.numpy as jnp
from jax.experimental import pallas as pl
from jax.experimental.pallas import tpu as pltpu


def _moe_body(x_ref, wg_ref, up_hbm, gate_hbm, down_hbm, out_ref,
              ubuf, gbuf, dbuf, sems):
    E = wg_ref.shape[1]

    xf = x_ref[...].astype(jnp.float32)
    logits = jnp.dot(xf, wg_ref[...], preferred_element_type=jnp.float32)
    m = jnp.max(logits, axis=-1, keepdims=True)
    p = jnp.exp(logits - m)
    g = p / jnp.sum(p, axis=-1, keepdims=True)
    col = jax.lax.broadcasted_iota(jnp.int32, g.shape, 1)
    i1 = jnp.argmax(g, axis=-1)[:, None]
    oh1 = col == i1
    i2 = jnp.argmax(jnp.where(oh1, -1.0, g), axis=-1)[:, None]
    oh2 = col == i2
    w = jnp.where(oh1 | oh2, g, 0.0)

    def copies(e):
        slot = e % 3
        return (
            pltpu.make_async_copy(up_hbm.at[e], ubuf.at[slot], sems.at[0, slot]),
            pltpu.make_async_copy(gate_hbm.at[e], gbuf.at[slot], sems.at[1, slot]),
            pltpu.make_async_copy(down_hbm.at[e], dbuf.at[slot], sems.at[2, slot]),
        )

    for c in copies(0):
        c.start()
    for c in copies(1):
        c.start()
    for c in copies(2):
        c.start()

    xb = x_ref[...]
    dn = (((1,), (1,)), ((), ()))
    H = ubuf.shape[1]
    HC = 512
    NC = H // HC
    wes = [jnp.sum(jnp.where(col == e, w, 0.0), axis=1, keepdims=True)
           for e in range(E)]
    acc = None
    pend = None
    for k in range(E * NC):
        e, c = divmod(k, NC)
        slot = e % 3
        cs = c * HC
        if c == 0:
            cu, cg, cd = copies(e)
            cu.wait()
            cg.wait()
            cd.wait()
        gate_c = jax.lax.dot_general(xb, gbuf[slot, cs:cs + HC, :], dn,
                                     preferred_element_type=jnp.float32)
        silu_c = gate_c * jax.nn.sigmoid(gate_c)
        up_c = jax.lax.dot_general(xb, ubuf[slot, cs:cs + HC, :], dn,
                                   preferred_element_type=jnp.float32)
        hidden_c = (silu_c * up_c).astype(jnp.bfloat16)
        if pend is not None:
            ph, pslot, pcs, pe = pend
            y_c = jax.lax.dot_general(ph, dbuf[pslot, :, pcs:pcs + HC], dn,
                                      preferred_element_type=jnp.float32)
            contrib = wes[pe] * y_c
            acc = contrib if acc is None else acc + contrib
        pend = (hidden_c, slot, cs, e)
        if c == 0 and e >= 1 and e + 2 < E:
            for cp in copies(e + 2):
                cp.start()
    ph, pslot, pcs, pe = pend
    y_c = jax.lax.dot_general(ph, dbuf[pslot, :, pcs:pcs + HC], dn,
                              preferred_element_type=jnp.float32)
    acc = acc + wes[pe] * y_c

    out_ref[...] = acc


@jax.jit
def kernel(x, W_gate, up_proj, gate_proj, down_proj):
    S, D = x.shape
    E, H, _ = up_proj.shape
    return pl.pallas_call(
        _moe_body,
        in_specs=[
            pl.BlockSpec(memory_space=pltpu.VMEM),
            pl.BlockSpec(memory_space=pltpu.VMEM),
            pl.BlockSpec(memory_space=pl.ANY),
            pl.BlockSpec(memory_space=pl.ANY),
            pl.BlockSpec(memory_space=pl.ANY),
        ],
        out_specs=pl.BlockSpec(memory_space=pltpu.VMEM),
        out_shape=jax.ShapeDtypeStruct((S, D), jnp.float32),
        scratch_shapes=[
            pltpu.VMEM((3, H, D), jnp.bfloat16),
            pltpu.VMEM((3, H, D), jnp.bfloat16),
            pltpu.VMEM((3, D, H), jnp.bfloat16),
            pltpu.SemaphoreType.DMA((3, 3)),
        ],
    )(x, W_gate, up_proj, gate_proj, down_proj)
